# baseline (speedup 1.0000x reference)
import jax
import jax.numpy as jnp
from jax.experimental import pallas as pl
from jax.experimental.pallas import tpu as pltpu

_Q = 1024
_K = 100000
_D = 32
_T = 10
_THRESHOLD = 1.0 / 3.0
_KBP = 2048
_NK = (_K // 4 + _KBP - 1) // _KBP
_BIG = 3.0e38
_PADV = 1.0e30
_L = 128


def _knn_kernel(ts_ref, xinit_ref, zs_ref, w_ref, xt_ref, out_ref,
                xa_scr, xs2_scr, m1_scr, m2_scr):
    pid = pl.program_id(0)

    @pl.when(pid == 0)
    def _prologue():
        x = xinit_ref[...]
        w = w_ref[...]
        for i in range(_T - 1):
            dt = ts_ref[i + 1] - ts_ref[i]
            x = (x + jnp.dot(x, w, preferred_element_type=jnp.float32) * dt
                 + jnp.sqrt(jnp.abs(dt)) * zs_ref[:, i * _D:(i + 1) * _D])
        xs2_scr[...] = jnp.sum(x * x, axis=1, keepdims=True)
        xa_scr[...] = jnp.concatenate(
            [-2.0 * x, jnp.ones((_Q, 1), jnp.float32)],
            axis=1).astype(jnp.bfloat16)
        m1_scr[...] = jnp.full((_Q, _L), _BIG, jnp.float32)
        m2_scr[...] = jnp.full((_Q, _L), _BIG, jnp.float32)

    xa = xa_scr[...]
    row = jax.lax.broadcasted_iota(jnp.int32, (_KBP, 1), 0) + pid * _KBP
    rmask = row < _K // 4
    pm = jnp.where(rmask, xt_ref[...], 0.0)
    li = jax.lax.broadcasted_iota(jnp.int32, (4 * _D, 4), 0)
    ci = jax.lax.broadcasted_iota(jnp.int32, (4 * _D, 4), 1)
    sel = (li // _D == ci).astype(jnp.float32)
    xt2g = jax.lax.dot_general(pm * pm, sel, (((1,), (0,)), ((), ())),
                               preferred_element_type=jnp.float32)
    xt2g = jnp.where(rmask, xt2g, _PADV).astype(jnp.bfloat16)
    pb = pm.astype(jnp.bfloat16)

    m1 = m1_scr[...]
    m2 = m2_scr[...]
    for c in range(4):
        xta = jnp.concatenate(
            [pb[:, c * _D:(c + 1) * _D], xt2g[:, c:c + 1]], axis=1)
        for j in range(_KBP // _L):
            v = jax.lax.dot_general(xa, xta[j * _L:(j + 1) * _L, :],
                                    (((1,), (1,)), ((), ())),
                                    preferred_element_type=jnp.float32)
            hi = jnp.maximum(m1, v)
            m1 = jnp.minimum(m1, v)
            m2 = jnp.minimum(m2, hi)
    m1_scr[...] = m1
    m2_scr[...] = m2

    @pl.when(pid == _NK - 1)
    def _epilogue():
        M1 = m1_scr[...]
        M2 = m2_scr[...]
        b1 = jnp.min(M1, axis=1, keepdims=True)
        eq = M1 == b1
        cnt = jnp.sum(eq.astype(jnp.float32), axis=1, keepdims=True)
        c_m1 = jnp.min(jnp.where(eq, _BIG, M1), axis=1, keepdims=True)
        c_m2 = jnp.min(jnp.where(eq, M2, _BIG), axis=1, keepdims=True)
        b2 = jnp.where(cnt > 1.0, b1, jnp.minimum(c_m1, c_m2))
        xs2 = xs2_scr[...]
        d0 = jnp.sqrt(jnp.maximum(b1 + xs2, 0.0))
        d1 = jnp.sqrt(jnp.maximum(b2 + xs2, 0.0))
        mem = (d0 < _THRESHOLD * d1).astype(jnp.float32)
        out_ref[...] = jnp.sum(mem, keepdims=True).reshape(1, 1) / jnp.float32(_Q)


def kernel(X_train, ts, x_init, zs, W):
    out = pl.pallas_call(
        _knn_kernel,
        grid=(_NK,),
        in_specs=[
            pl.BlockSpec(memory_space=pltpu.SMEM),
            pl.BlockSpec((_Q, _D), lambda k: (0, 0)),
            pl.BlockSpec((_Q, (_T - 1) * _D), lambda k: (0, 0)),
            pl.BlockSpec((_D, _D), lambda k: (0, 0)),
            pl.BlockSpec((_KBP, 4 * _D), lambda k: (k, 0)),
        ],
        out_specs=pl.BlockSpec((1, 1), lambda k: (0, 0)),
        out_shape=jax.ShapeDtypeStruct((1, 1), jnp.float32),
        scratch_shapes=[
            pltpu.VMEM((_Q, _D + 1), jnp.bfloat16),
            pltpu.VMEM((_Q, 1), jnp.float32),
            pltpu.VMEM((_Q, _L), jnp.float32),
            pltpu.VMEM((_Q, _L), jnp.float32),
        ],
        compiler_params=pltpu.CompilerParams(
            dimension_semantics=("arbitrary",),
        ),
    )(ts, x_init, zs.reshape(_Q, (_T - 1) * _D), W,
      X_train.reshape(_K // 4, 4 * _D))
    return out[0, 0]


# speedup vs baseline: 1.1866x; 1.1866x over previous
import jax
import jax.numpy as jnp
from jax.experimental import pallas as pl
from jax.experimental.pallas import tpu as pltpu

_Q = 1024
_K = 100000
_D = 32
_T = 10
_THRESHOLD = 1.0 / 3.0
_KB = 8192
_NK = (_K + _KB - 1) // _KB
_BIG = 3.0e38
_PADV = 1.0e30
_L = 128


def _knn_kernel(ts_ref, xinit_ref, zs_ref, w_ref, xt_ref, out_ref,
                xa_scr, xs2_scr, m1_scr, m2_scr):
    pid = pl.program_id(0)

    @pl.when(pid == 0)
    def _prologue():
        x = xinit_ref[...]
        w = w_ref[...]
        for i in range(_T - 1):
            dt = ts_ref[i + 1] - ts_ref[i]
            x = (x + jnp.dot(x, w, preferred_element_type=jnp.float32) * dt
                 + jnp.sqrt(jnp.abs(dt)) * zs_ref[:, i * _D:(i + 1) * _D])
        xs2_scr[...] = jnp.sum(x * x, axis=1, keepdims=True)
        xa_scr[...] = jnp.concatenate(
            [-2.0 * x, jnp.ones((_Q, 1), jnp.float32)],
            axis=1).astype(jnp.bfloat16)
        m1_scr[...] = jnp.full((_Q, _L), _BIG, jnp.float32)
        m2_scr[...] = jnp.full((_Q, _L), _BIG, jnp.float32)

    xa = xa_scr[...]
    row = jax.lax.broadcasted_iota(jnp.int32, (_KB, 1), 0) + pid * _KB
    rmask = row < _K
    xtf = jnp.where(rmask, xt_ref[...], 0.0)
    xt2m = jax.lax.dot_general(xtf * xtf, jnp.ones((_D, 1), jnp.float32),
                               (((1,), (0,)), ((), ())),
                               preferred_element_type=jnp.float32)
    xt2f = jnp.where(rmask, xt2m, _PADV)
    xta = jnp.concatenate(
        [xtf.astype(jnp.bfloat16), xt2f.astype(jnp.bfloat16)], axis=1)

    m1 = m1_scr[...]
    m2 = m2_scr[...]
    for j in range(_KB // _L):
        v = jax.lax.dot_general(xa, xta[j * _L:(j + 1) * _L, :],
                                (((1,), (1,)), ((), ())),
                                preferred_element_type=jnp.float32)
        hi = jnp.maximum(m1, v)
        m1 = jnp.minimum(m1, v)
        m2 = jnp.minimum(m2, hi)
    m1_scr[...] = m1
    m2_scr[...] = m2

    @pl.when(pid == _NK - 1)
    def _epilogue():
        M1 = m1_scr[...]
        M2 = m2_scr[...]
        b1 = jnp.min(M1, axis=1, keepdims=True)
        eq = M1 == b1
        cnt = jnp.sum(eq.astype(jnp.float32), axis=1, keepdims=True)
        c_m1 = jnp.min(jnp.where(eq, _BIG, M1), axis=1, keepdims=True)
        c_m2 = jnp.min(jnp.where(eq, M2, _BIG), axis=1, keepdims=True)
        b2 = jnp.where(cnt > 1.0, b1, jnp.minimum(c_m1, c_m2))
        xs2 = xs2_scr[...]
        d0 = jnp.sqrt(jnp.maximum(b1 + xs2, 0.0))
        d1 = jnp.sqrt(jnp.maximum(b2 + xs2, 0.0))
        mem = (d0 < _THRESHOLD * d1).astype(jnp.float32)
        out_ref[...] = jnp.sum(mem, keepdims=True).reshape(1, 1) / jnp.float32(_Q)


def kernel(X_train, ts, x_init, zs, W):
    out = pl.pallas_call(
        _knn_kernel,
        grid=(_NK,),
        in_specs=[
            pl.BlockSpec(memory_space=pltpu.SMEM),
            pl.BlockSpec((_Q, _D), lambda k: (0, 0)),
            pl.BlockSpec((_Q, (_T - 1) * _D), lambda k: (0, 0)),
            pl.BlockSpec((_D, _D), lambda k: (0, 0)),
            pl.BlockSpec((_KB, _D), lambda k: (k, 0)),
        ],
        out_specs=pl.BlockSpec((1, 1), lambda k: (0, 0)),
        out_shape=jax.ShapeDtypeStruct((1, 1), jnp.float32),
        scratch_shapes=[
            pltpu.VMEM((_Q, _D + 1), jnp.bfloat16),
            pltpu.VMEM((_Q, 1), jnp.float32),
            pltpu.VMEM((_Q, _L), jnp.float32),
            pltpu.VMEM((_Q, _L), jnp.float32),
        ],
        compiler_params=pltpu.CompilerParams(
            dimension_semantics=("arbitrary",),
        ),
    )(ts, x_init, zs.reshape(_Q, (_T - 1) * _D), W, X_train)
    return out[0, 0]


# speedup vs baseline: 1.2874x; 1.0849x over previous
import jax
import jax.numpy as jnp
from jax.experimental import pallas as pl
from jax.experimental.pallas import tpu as pltpu

_Q = 1024
_K = 100000
_D = 32
_T = 10
_THRESHOLD = 1.0 / 3.0
_KB = 8192
_NK = (_K + _KB - 1) // _KB
_BIG = 3.0e38
_PADV = 1.0e30
_L = 128


def _knn_kernel(ts_ref, xinit_ref, zs_ref, w_ref, xt_ref, out_ref,
                xa_scr, xs2_scr, m1_scr, m2_scr):
    pid = pl.program_id(0)

    @pl.when(pid == 0)
    def _prologue():
        x = xinit_ref[...]
        w = w_ref[...]
        for i in range(_T - 1):
            dt = ts_ref[i + 1] - ts_ref[i]
            x = (x + jnp.dot(x, w, preferred_element_type=jnp.float32) * dt
                 + jnp.sqrt(jnp.abs(dt)) * zs_ref[:, i * _D:(i + 1) * _D])
        xs2_scr[...] = jnp.sum(x * x, axis=1, keepdims=True)
        xa_scr[...] = jnp.concatenate(
            [-2.0 * x, jnp.ones((_Q, 1), jnp.float32)],
            axis=1).astype(jnp.bfloat16)
        m1_scr[...] = jnp.full((_Q, _L), _BIG, jnp.float32)
        m2_scr[...] = jnp.full((_Q, _L), _BIG, jnp.float32)

    xa = xa_scr[...]
    row = jax.lax.broadcasted_iota(jnp.int32, (_KB, 1), 0) + pid * _KB
    rmask = row < _K
    xtf = jnp.where(rmask, xt_ref[...], 0.0)
    xt2f = jnp.where(rmask,
                     jnp.sum(xtf * xtf, axis=1, keepdims=True), _PADV)
    xta = jnp.concatenate(
        [xtf.astype(jnp.bfloat16), xt2f.astype(jnp.bfloat16)], axis=1)

    m1 = m1_scr[...]
    m2 = m2_scr[...]
    for j in range(_KB // _L):
        v = jax.lax.dot_general(xa, xta[j * _L:(j + 1) * _L, :],
                                (((1,), (1,)), ((), ())),
                                preferred_element_type=jnp.float32)
        hi = jnp.maximum(m1, v)
        m1 = jnp.minimum(m1, v)
        m2 = jnp.minimum(m2, hi)
    m1_scr[...] = m1
    m2_scr[...] = m2

    @pl.when(pid == _NK - 1)
    def _epilogue():
        M1 = m1_scr[...]
        M2 = m2_scr[...]
        b1 = jnp.min(M1, axis=1, keepdims=True)
        eq = M1 == b1
        cnt = jnp.sum(eq.astype(jnp.float32), axis=1, keepdims=True)
        c_m1 = jnp.min(jnp.where(eq, _BIG, M1), axis=1, keepdims=True)
        c_m2 = jnp.min(jnp.where(eq, M2, _BIG), axis=1, keepdims=True)
        b2 = jnp.where(cnt > 1.0, b1, jnp.minimum(c_m1, c_m2))
        xs2 = xs2_scr[...]
        d0 = jnp.sqrt(jnp.maximum(b1 + xs2, 0.0))
        d1 = jnp.sqrt(jnp.maximum(b2 + xs2, 0.0))
        mem = (d0 < _THRESHOLD * d1).astype(jnp.float32)
        out_ref[...] = jnp.sum(mem, keepdims=True).reshape(1, 1) / jnp.float32(_Q)


def kernel(X_train, ts, x_init, zs, W):
    out = pl.pallas_call(
        _knn_kernel,
        grid=(_NK,),
        in_specs=[
            pl.BlockSpec(memory_space=pltpu.SMEM),
            pl.BlockSpec((_Q, _D), lambda k: (0, 0)),
            pl.BlockSpec((_Q, (_T - 1) * _D), lambda k: (0, 0)),
            pl.BlockSpec((_D, _D), lambda k: (0, 0)),
            pl.BlockSpec((_KB, _D), lambda k: (k, 0)),
        ],
        out_specs=pl.BlockSpec((1, 1), lambda k: (0, 0)),
        out_shape=jax.ShapeDtypeStruct((1, 1), jnp.float32),
        scratch_shapes=[
            pltpu.VMEM((_Q, _D + 1), jnp.bfloat16),
            pltpu.VMEM((_Q, 1), jnp.float32),
            pltpu.VMEM((_Q, _L), jnp.float32),
            pltpu.VMEM((_Q, _L), jnp.float32),
        ],
        compiler_params=pltpu.CompilerParams(
            dimension_semantics=("arbitrary",),
        ),
    )(ts, x_init, zs.reshape(_Q, (_T - 1) * _D), W, X_train)
    return out[0, 0]
